# Initial kernel scaffold; baseline (speedup 1.0000x reference)
import jax
import jax.numpy as jnp
from jax.experimental import pallas as pl


def kernel(x, edge_index, edge_attr, batch, W_in, b_in, We, be, W1, b1, W2, b2, eps, ln_g, ln_b, Wr1, br1, Wr2, br2, Wr3, br3):
    raise NotImplementedError("write your pallas kernel here")


# speedup vs baseline: 2.5089x; 2.5089x over previous
import functools

import jax
import jax.numpy as jnp
from jax import lax
from jax.experimental import pallas as pl
from jax.experimental.pallas import tpu as pltpu
from jax.experimental.pallas import tpu_sc as plsc

N_NODES = 10000
N_EDGES = 320000
D = 128
D_EDGE = 16
N_LAYERS = 4
N_GRAPHS = 128

NC = 2
NS = 16
NW = NC * NS
EPW = N_EDGES // NW
CH = 128
NCHUNK = EPW // CH
TAIL = EPW - NCHUNK * CH

ROW_BLK = 1000
N_ROW_BLKS = N_NODES // ROW_BLK
E_BLK = 2000
N_E_BLKS = N_EDGES // E_BLK


def _proj_in_body(x_ref, w_ref, b_ref, o_ref):
    o_ref[...] = (
        jnp.dot(x_ref[...], w_ref[...], preferred_element_type=jnp.float32)
        + b_ref[...]
    )


def _proj_in(x, W_in, b_in):
    return pl.pallas_call(
        _proj_in_body,
        grid=(N_ROW_BLKS,),
        in_specs=[
            pl.BlockSpec((ROW_BLK, D), lambda i: (i, 0)),
            pl.BlockSpec((D, D), lambda i: (0, 0)),
            pl.BlockSpec((1, D), lambda i: (0, 0)),
        ],
        out_specs=pl.BlockSpec((ROW_BLK, D), lambda i: (i, 0)),
        out_shape=jax.ShapeDtypeStruct((N_NODES, D), jnp.float32),
    )(x, W_in, b_in.reshape(1, D))


def _eproj_body(ea_ref, we_ref, be_ref, o_ref):
    o_ref[0] = (
        jnp.dot(ea_ref[...], we_ref[0], preferred_element_type=jnp.float32)
        + be_ref[0]
    )


def _eproj(edge_attr, We, be):
    return pl.pallas_call(
        _eproj_body,
        grid=(N_LAYERS, N_E_BLKS),
        in_specs=[
            pl.BlockSpec((E_BLK, D_EDGE), lambda l, i: (i, 0)),
            pl.BlockSpec((1, D_EDGE, D), lambda l, i: (l, 0, 0)),
            pl.BlockSpec((1, 1, D), lambda l, i: (l, 0, 0)),
        ],
        out_specs=pl.BlockSpec((1, E_BLK, D), lambda l, i: (l, i, 0)),
        out_shape=jax.ShapeDtypeStruct((N_LAYERS, N_EDGES, D), jnp.float32),
    )(edge_attr, We, be.reshape(N_LAYERS, 1, D))


def _node_update_body(eps_ref, h_ref, agg_ref, w1_ref, b1_ref, w2_ref, b2_ref,
                      g_ref, b_ref, o_ref):
    h = h_ref[...]
    u = eps_ref[0] * h + agg_ref[0] + agg_ref[1]
    t = jnp.maximum(
        jnp.dot(u, w1_ref[...], preferred_element_type=jnp.float32)
        + b1_ref[...], 0.0)
    v = (jnp.dot(t, w2_ref[...], preferred_element_type=jnp.float32)
         + b2_ref[...])
    mu = jnp.mean(v, axis=-1, keepdims=True)
    var = jnp.mean((v - mu) ** 2, axis=-1, keepdims=True)
    ln = (v - mu) * lax.rsqrt(var + 1e-5) * g_ref[...] + b_ref[...]
    o_ref[...] = h + ln


def _node_update(h, agg2, epsp1, W1l, b1l, W2l, b2l, gl, bl):
    return pl.pallas_call(
        _node_update_body,
        grid=(N_ROW_BLKS,),
        in_specs=[
            pl.BlockSpec(memory_space=pltpu.SMEM),
            pl.BlockSpec((ROW_BLK, D), lambda i: (i, 0)),
            pl.BlockSpec((NC, ROW_BLK, D), lambda i: (0, i, 0)),
            pl.BlockSpec((D, D), lambda i: (0, 0)),
            pl.BlockSpec((1, D), lambda i: (0, 0)),
            pl.BlockSpec((D, D), lambda i: (0, 0)),
            pl.BlockSpec((1, D), lambda i: (0, 0)),
            pl.BlockSpec((1, D), lambda i: (0, 0)),
            pl.BlockSpec((1, D), lambda i: (0, 0)),
        ],
        out_specs=pl.BlockSpec((ROW_BLK, D), lambda i: (i, 0)),
        out_shape=jax.ShapeDtypeStruct((N_NODES, D), jnp.float32),
    )(epsp1, h, agg2, W1l, b1l.reshape(1, D), W2l, b2l.reshape(1, D),
      gl.reshape(1, D), bl.reshape(1, D))


def _pool_head_body(h_ref, batch_ref, wr1_ref, br1_ref, wr2_ref, br2_ref,
                    wr3_ref, br3_ref, o_ref, acc_ref, cnt_ref):
    i = pl.program_id(0)

    @pl.when(i == 0)
    def _():
        acc_ref[...] = jnp.zeros_like(acc_ref)
        cnt_ref[...] = jnp.zeros_like(cnt_ref)

    bb = batch_ref[0, 0, :]
    gids = lax.broadcasted_iota(jnp.int32, (N_GRAPHS, ROW_BLK), 0)
    onehot = (bb[None, :] == gids).astype(jnp.float32)
    acc_ref[...] += jnp.dot(onehot, h_ref[...],
                            preferred_element_type=jnp.float32)
    cnt_ref[...] += jnp.dot(onehot, jnp.ones((ROW_BLK, D), jnp.float32),
                            preferred_element_type=jnp.float32)

    @pl.when(i == N_ROW_BLKS - 1)
    def _():
        g = acc_ref[...] / jnp.maximum(cnt_ref[...], 1.0)
        a = jnp.maximum(
            jnp.dot(g, wr1_ref[...], preferred_element_type=jnp.float32)
            + br1_ref[...], 0.0)
        b = jnp.maximum(
            jnp.dot(a, wr2_ref[...], preferred_element_type=jnp.float32)
            + br2_ref[...], 0.0)
        o_ref[...] = (
            jnp.dot(b, wr3_ref[...], preferred_element_type=jnp.float32)
            + br3_ref[...])


def _pool_head(h, batch_i32, Wr1, br1, Wr2, br2, Wr3, br3):
    Wr3p = jnp.pad(Wr3, ((0, 0), (0, D - Wr3.shape[1])))
    br3p = jnp.pad(br3, (0, D - br3.shape[0])).reshape(1, D)
    out = pl.pallas_call(
        _pool_head_body,
        grid=(N_ROW_BLKS,),
        in_specs=[
            pl.BlockSpec((ROW_BLK, D), lambda i: (i, 0)),
            pl.BlockSpec((1, 1, ROW_BLK), lambda i: (i, 0, 0)),
            pl.BlockSpec((D, D), lambda i: (0, 0)),
            pl.BlockSpec((1, D), lambda i: (0, 0)),
            pl.BlockSpec((D, 64), lambda i: (0, 0)),
            pl.BlockSpec((1, 64), lambda i: (0, 0)),
            pl.BlockSpec((64, D), lambda i: (0, 0)),
            pl.BlockSpec((1, D), lambda i: (0, 0)),
        ],
        out_specs=pl.BlockSpec((N_GRAPHS, D), lambda i: (0, 0)),
        out_shape=jax.ShapeDtypeStruct((N_GRAPHS, D), jnp.float32),
        scratch_shapes=[
            pltpu.VMEM((N_GRAPHS, D), jnp.float32),
            pltpu.VMEM((N_GRAPHS, D), jnp.float32),
        ],
    )(h, batch_i32.reshape(N_ROW_BLKS, 1, ROW_BLK), Wr1, br1.reshape(1, D),
      Wr2, br2.reshape(1, 64), Wr3p, br3p)
    return out[:, :1]


@functools.lru_cache(maxsize=None)
def _make_edge_agg(layer):
    mesh = plsc.VectorSubcoreMesh(core_axis_name="c", subcore_axis_name="s",
                                  num_cores=NC)

    @functools.partial(
        pl.kernel,
        mesh=mesh,
        out_type=jax.ShapeDtypeStruct((NC, N_NODES, D), jnp.float32),
        scratch_types=[
            pltpu.VMEM((CH,), jnp.int32),
            pltpu.VMEM((CH,), jnp.int32),
            pltpu.VMEM((CH, D), jnp.float32),
            pltpu.VMEM((CH, D), jnp.float32),
            pltpu.VMEM((TAIL,), jnp.int32),
            pltpu.VMEM((TAIL,), jnp.int32),
            pltpu.VMEM((TAIL, D), jnp.float32),
            pltpu.VMEM((TAIL, D), jnp.float32),
            pltpu.VMEM_SHARED((N_NODES, D), jnp.float32),
            pltpu.SemaphoreType.DMA,
        ],
    )
    def edge_agg(h_hbm, e_hbm, src_hbm, dst_hbm, z_hbm, out_hbm,
                 sidx, didx, hbuf, ebuf, sidx_t, didx_t, hbuf_t, ebuf_t,
                 aggs, sem):
        cid = lax.axis_index("c")
        sid = lax.axis_index("s")
        wid = cid * NS + sid

        @pl.when(sid == 0)
        def _():
            pltpu.sync_copy(z_hbm, aggs)

        plsc.subcore_barrier()

        def do_chunk(base, n, si, di, hb, eb):
            pltpu.sync_copy(src_hbm.at[pl.ds(base, n)], si)
            pltpu.sync_copy(dst_hbm.at[pl.ds(base, n)], di)
            pltpu.async_copy(h_hbm.at[si], hb, sem).wait()
            pltpu.sync_copy(e_hbm.at[layer, pl.ds(base, n)], eb)

            def row(j, carry):
                for k in range(D // 16):
                    sl = pl.ds(k * 16, 16)
                    v = hb[j, sl] + eb[j, sl]
                    hb[j, sl] = jnp.maximum(v, 0.0)
                return carry

            lax.fori_loop(0, n, row, 0)
            pltpu.sync_copy(hb, aggs.at[di], add=True)

        def chunk(i, carry):
            do_chunk(wid * EPW + i * CH, CH, sidx, didx, hbuf, ebuf)
            return carry

        lax.fori_loop(0, NCHUNK, chunk, 0)
        do_chunk(wid * EPW + NCHUNK * CH, TAIL, sidx_t, didx_t, hbuf_t, ebuf_t)

        plsc.subcore_barrier()

        @pl.when(sid == 0)
        def _():
            pltpu.sync_copy(aggs, out_hbm.at[cid])

    return edge_agg


def kernel(x, edge_index, edge_attr, batch, W_in, b_in, We, be, W1, b1, W2, b2,
           eps, ln_g, ln_b, Wr1, br1, Wr2, br2, Wr3, br3):
    src = edge_index[0].astype(jnp.int32)
    dst = edge_index[1].astype(jnp.int32)
    batch_i32 = batch.astype(jnp.int32)
    zeros = jnp.zeros((N_NODES, D), jnp.float32)

    h = _proj_in(x, W_in, b_in)
    e_all = _eproj(edge_attr, We, be)
    for l in range(N_LAYERS):
        agg2 = _make_edge_agg(l)(h, e_all, src, dst, zeros)
        epsp1 = (1.0 + eps[l]).reshape(1).astype(jnp.float32)
        h = _node_update(h, agg2, epsp1, W1[l], b1[l], W2[l], b2[l],
                         ln_g[l], ln_b[l])
    return _pool_head(h, batch_i32, Wr1, br1, Wr2, br2, Wr3, br3)


# speedup vs baseline: 3.8730x; 1.5437x over previous
import functools

import jax
import jax.numpy as jnp
from jax import lax
from jax.experimental import pallas as pl
from jax.experimental.pallas import tpu as pltpu
from jax.experimental.pallas import tpu_sc as plsc

N_NODES = 10000
N_EDGES = 320000
D = 128
D_EDGE = 16
N_LAYERS = 4
N_GRAPHS = 128

NC = 2
NS = 16
NW = NC * NS
EPW = N_EDGES // NW
CH = 80
NCHUNK = EPW // CH

ROW_BLK = 1000
N_ROW_BLKS = N_NODES // ROW_BLK
E_BLK = 2000
N_E_BLKS = N_EDGES // E_BLK


def _proj_in_body(x_ref, w_ref, b_ref, o_ref):
    o_ref[...] = (
        jnp.dot(x_ref[...], w_ref[...], preferred_element_type=jnp.float32)
        + b_ref[...]
    )


def _proj_in(x, W_in, b_in):
    return pl.pallas_call(
        _proj_in_body,
        grid=(N_ROW_BLKS,),
        in_specs=[
            pl.BlockSpec((ROW_BLK, D), lambda i: (i, 0)),
            pl.BlockSpec((D, D), lambda i: (0, 0)),
            pl.BlockSpec((1, D), lambda i: (0, 0)),
        ],
        out_specs=pl.BlockSpec((ROW_BLK, D), lambda i: (i, 0)),
        out_shape=jax.ShapeDtypeStruct((N_NODES, D), jnp.float32),
    )(x, W_in, b_in.reshape(1, D))


def _eproj_body(ea_ref, we_ref, be_ref, o_ref):
    o_ref[0] = (
        jnp.dot(ea_ref[...], we_ref[0], preferred_element_type=jnp.float32)
        + be_ref[0]
    )


def _eproj(edge_attr, We, be):
    return pl.pallas_call(
        _eproj_body,
        grid=(N_LAYERS, N_E_BLKS),
        in_specs=[
            pl.BlockSpec((E_BLK, D_EDGE), lambda l, i: (i, 0)),
            pl.BlockSpec((1, D_EDGE, D), lambda l, i: (l, 0, 0)),
            pl.BlockSpec((1, 1, D), lambda l, i: (l, 0, 0)),
        ],
        out_specs=pl.BlockSpec((1, E_BLK, D), lambda l, i: (l, i, 0)),
        out_shape=jax.ShapeDtypeStruct((N_LAYERS, N_EDGES, D), jnp.float32),
    )(edge_attr, We, be.reshape(N_LAYERS, 1, D))


def _node_update_body(eps_ref, h_ref, agg_ref, w1_ref, b1_ref, w2_ref, b2_ref,
                      g_ref, b_ref, o_ref):
    h = h_ref[...]
    u = eps_ref[0] * h + agg_ref[0] + agg_ref[1]
    t = jnp.maximum(
        jnp.dot(u, w1_ref[...], preferred_element_type=jnp.float32)
        + b1_ref[...], 0.0)
    v = (jnp.dot(t, w2_ref[...], preferred_element_type=jnp.float32)
         + b2_ref[...])
    mu = jnp.mean(v, axis=-1, keepdims=True)
    var = jnp.mean((v - mu) ** 2, axis=-1, keepdims=True)
    ln = (v - mu) * lax.rsqrt(var + 1e-5) * g_ref[...] + b_ref[...]
    o_ref[...] = h + ln


def _node_update(h, agg2, epsp1, W1l, b1l, W2l, b2l, gl, bl):
    return pl.pallas_call(
        _node_update_body,
        grid=(N_ROW_BLKS,),
        in_specs=[
            pl.BlockSpec(memory_space=pltpu.SMEM),
            pl.BlockSpec((ROW_BLK, D), lambda i: (i, 0)),
            pl.BlockSpec((NC, ROW_BLK, D), lambda i: (0, i, 0)),
            pl.BlockSpec((D, D), lambda i: (0, 0)),
            pl.BlockSpec((1, D), lambda i: (0, 0)),
            pl.BlockSpec((D, D), lambda i: (0, 0)),
            pl.BlockSpec((1, D), lambda i: (0, 0)),
            pl.BlockSpec((1, D), lambda i: (0, 0)),
            pl.BlockSpec((1, D), lambda i: (0, 0)),
        ],
        out_specs=pl.BlockSpec((ROW_BLK, D), lambda i: (i, 0)),
        out_shape=jax.ShapeDtypeStruct((N_NODES, D), jnp.float32),
    )(epsp1, h, agg2, W1l, b1l.reshape(1, D), W2l, b2l.reshape(1, D),
      gl.reshape(1, D), bl.reshape(1, D))


def _pool_head_body(h_ref, batch_ref, wr1_ref, br1_ref, wr2_ref, br2_ref,
                    wr3_ref, br3_ref, o_ref, acc_ref, cnt_ref):
    i = pl.program_id(0)

    @pl.when(i == 0)
    def _():
        acc_ref[...] = jnp.zeros_like(acc_ref)
        cnt_ref[...] = jnp.zeros_like(cnt_ref)

    bb = batch_ref[0, 0, :]
    gids = lax.broadcasted_iota(jnp.int32, (N_GRAPHS, ROW_BLK), 0)
    onehot = (bb[None, :] == gids).astype(jnp.float32)
    acc_ref[...] += jnp.dot(onehot, h_ref[...],
                            preferred_element_type=jnp.float32)
    cnt_ref[...] += jnp.dot(onehot, jnp.ones((ROW_BLK, D), jnp.float32),
                            preferred_element_type=jnp.float32)

    @pl.when(i == N_ROW_BLKS - 1)
    def _():
        g = acc_ref[...] / jnp.maximum(cnt_ref[...], 1.0)
        a = jnp.maximum(
            jnp.dot(g, wr1_ref[...], preferred_element_type=jnp.float32)
            + br1_ref[...], 0.0)
        b = jnp.maximum(
            jnp.dot(a, wr2_ref[...], preferred_element_type=jnp.float32)
            + br2_ref[...], 0.0)
        o_ref[...] = (
            jnp.dot(b, wr3_ref[...], preferred_element_type=jnp.float32)
            + br3_ref[...])


def _pool_head(h, batch_i32, Wr1, br1, Wr2, br2, Wr3, br3):
    Wr3p = jnp.pad(Wr3, ((0, 0), (0, D - Wr3.shape[1])))
    br3p = jnp.pad(br3, (0, D - br3.shape[0])).reshape(1, D)
    out = pl.pallas_call(
        _pool_head_body,
        grid=(N_ROW_BLKS,),
        in_specs=[
            pl.BlockSpec((ROW_BLK, D), lambda i: (i, 0)),
            pl.BlockSpec((1, 1, ROW_BLK), lambda i: (i, 0, 0)),
            pl.BlockSpec((D, D), lambda i: (0, 0)),
            pl.BlockSpec((1, D), lambda i: (0, 0)),
            pl.BlockSpec((D, 64), lambda i: (0, 0)),
            pl.BlockSpec((1, 64), lambda i: (0, 0)),
            pl.BlockSpec((64, D), lambda i: (0, 0)),
            pl.BlockSpec((1, D), lambda i: (0, 0)),
        ],
        out_specs=pl.BlockSpec((N_GRAPHS, D), lambda i: (0, 0)),
        out_shape=jax.ShapeDtypeStruct((N_GRAPHS, D), jnp.float32),
        scratch_shapes=[
            pltpu.VMEM((N_GRAPHS, D), jnp.float32),
            pltpu.VMEM((N_GRAPHS, D), jnp.float32),
        ],
    )(h, batch_i32.reshape(N_ROW_BLKS, 1, ROW_BLK), Wr1, br1.reshape(1, D),
      Wr2, br2.reshape(1, 64), Wr3p, br3p)
    return out[:, :1]


@functools.lru_cache(maxsize=None)
def _make_edge_agg(layer):
    mesh = plsc.VectorSubcoreMesh(core_axis_name="c", subcore_axis_name="s",
                                  num_cores=NC)

    @functools.partial(
        pl.kernel,
        mesh=mesh,
        out_type=jax.ShapeDtypeStruct((NC, N_NODES, D), jnp.float32),
        scratch_types=[
            pltpu.VMEM((2, CH), jnp.int32),
            pltpu.VMEM((2, CH), jnp.int32),
            pltpu.VMEM((CH, D), jnp.float32),
            pltpu.VMEM((CH, D), jnp.float32),
            pltpu.VMEM((CH, D), jnp.float32),
            pltpu.VMEM((CH, D), jnp.float32),
            pltpu.VMEM_SHARED((N_NODES, D), jnp.float32),
            pltpu.SemaphoreType.DMA,
            pltpu.SemaphoreType.DMA,
            pltpu.SemaphoreType.DMA,
            pltpu.SemaphoreType.DMA,
            pltpu.SemaphoreType.DMA,
            pltpu.SemaphoreType.DMA,
            pltpu.SemaphoreType.DMA,
            pltpu.SemaphoreType.DMA,
        ],
    )
    def edge_agg(h_hbm, e_hbm, src_hbm, dst_hbm, z_hbm, out_hbm,
                 sbuf, dbuf, hbuf0, hbuf1, ebuf0, ebuf1, aggs,
                 semg0, semg1, seme0, seme1, semi0, semi1, sems0, sems1):
        cid = lax.axis_index("c")
        sid = lax.axis_index("s")
        wid = cid * NS + sid
        ebase = wid * EPW

        hb = (hbuf0, hbuf1)
        eb = (ebuf0, ebuf1)
        semg = (semg0, semg1)
        seme = (seme0, seme1)
        semi = (semi0, semi1)
        sems = (sems0, sems1)

        @pl.when(sid == 0)
        def _():
            pltpu.sync_copy(z_hbm, aggs)

        plsc.subcore_barrier()

        def sidx_copy(i, b):
            return pltpu.make_async_copy(
                src_hbm.at[pl.ds(ebase + i * CH, CH)], sbuf.at[b], sems[b])

        def start_main(i, b):
            pltpu.make_async_copy(
                h_hbm.at[sbuf.at[b]], hb[b], semg[b]).start()
            pltpu.make_async_copy(
                e_hbm.at[layer, pl.ds(ebase + i * CH, CH)], eb[b],
                seme[b]).start()
            pltpu.make_async_copy(
                dst_hbm.at[pl.ds(ebase + i * CH, CH)], dbuf.at[b],
                semi[b]).start()

        def finish_main(i, b):
            pltpu.make_async_copy(h_hbm.at[sbuf.at[b]], hb[b], semg[b]).wait()
            pltpu.make_async_copy(
                e_hbm.at[layer, pl.ds(ebase + i * CH, CH)], eb[b],
                seme[b]).wait()
            pltpu.make_async_copy(
                dst_hbm.at[pl.ds(ebase + i * CH, CH)], dbuf.at[b],
                semi[b]).wait()

        def compute(hbr, ebr):
            def row(j, carry):
                for k in range(D // 16):
                    sl = pl.ds(k * 16, 16)
                    v = hbr[j, sl] + ebr[j, sl]
                    hbr[j, sl] = jnp.maximum(v, 0.0)
                return carry

            lax.fori_loop(0, CH, row, 0)

        def body(i, b):
            sidx_copy(i + 1, 1 - b).wait()
            finish_main(i, b)

            @pl.when(i + 2 < NCHUNK)
            def _():
                sidx_copy(i + 2, b).start()

            start_main(i + 1, 1 - b)
            compute(hb[b], eb[b])
            pltpu.sync_copy(hb[b], aggs.at[dbuf.at[b]], add=True)

        pltpu.sync_copy(src_hbm.at[pl.ds(ebase, CH)], sbuf.at[0])
        sidx_copy(1, 1).start()
        start_main(0, 0)

        def step2(i2, carry):
            body(i2 * 2, 0)
            body(i2 * 2 + 1, 1)
            return carry

        lax.fori_loop(0, NCHUNK // 2, step2, 0)

        finish_main(NCHUNK - 1, 0)
        compute(hb[0], eb[0])
        pltpu.sync_copy(hb[0], aggs.at[dbuf.at[0]], add=True)

        plsc.subcore_barrier()

        @pl.when(sid == 0)
        def _():
            pltpu.sync_copy(aggs, out_hbm.at[cid])

    return edge_agg


def kernel(x, edge_index, edge_attr, batch, W_in, b_in, We, be, W1, b1, W2, b2,
           eps, ln_g, ln_b, Wr1, br1, Wr2, br2, Wr3, br3):
    src = edge_index[0].astype(jnp.int32)
    dst = edge_index[1].astype(jnp.int32)
    batch_i32 = batch.astype(jnp.int32)
    zeros = jnp.zeros((N_NODES, D), jnp.float32)

    h = _proj_in(x, W_in, b_in)
    e_all = _eproj(edge_attr, We, be)
    for l in range(N_LAYERS):
        agg2 = _make_edge_agg(l)(h, e_all, src, dst, zeros)
        epsp1 = (1.0 + eps[l]).reshape(1).astype(jnp.float32)
        h = _node_update(h, agg2, epsp1, W1[l], b1[l], W2[l], b2[l],
                         ln_g[l], ln_b[l])
    return _pool_head(h, batch_i32, Wr1, br1, Wr2, br2, Wr3, br3)
